# baseline (speedup 1.0000x reference)
import jax
import jax.numpy as jnp
from jax import lax
from jax.experimental import pallas as pl
from jax.experimental.pallas import tpu as pltpu
from jax.experimental.pallas import tpu_sc as plsc

LIMIT = 384
NC, NS, LANES = 2, 16, 16
NW = NC * NS
CHUNK = 64
RGRP = 8
RSUB = 4
NBUF = 2


def _score_body(ids_hbm, taps_hbm, table_hbm, g_hbm,
                idx_v, rows_v, g_v, taps_v, sem0, sem1):
    D = table_hbm.shape[1]
    S = ids_hbm.shape[1]
    ppw = g_v.shape[1] * 8
    nch = ppw // CHUNK
    nslices = D // LANES
    sems = (sem0, sem1)
    wid = lax.axis_index("s") * NC + lax.axis_index("c")
    base = wid * ppw
    b_row = base // S
    s0 = base - b_row * S

    pltpu.sync_copy(ids_hbm.at[b_row, pl.ds(s0, ppw)], idx_v)

    def start_gather(cc, buf):
        pltpu.async_copy(
            table_hbm.at[idx_v.at[pl.ds(cc * CHUNK, CHUNK)]],
            rows_v.at[buf], sems[buf])

    def wait_gather(buf):
        pltpu.make_async_copy(
            table_hbm.at[idx_v.at[pl.ds(0, CHUNK)]],
            rows_v.at[buf], sems[buf]).wait()

    for b in range(NBUF):
        start_gather(b, b)
    pltpu.sync_copy(taps_hbm, taps_v)

    DUNROLL = 2

    def compute_chunk(cc, buf):
        @plsc.parallel_loop(0, CHUNK // RGRP, unroll=2)
        def grp_body(gi):
            zero = jnp.zeros((LANES,), jnp.float32)
            for h in range(RGRP // RSUB):
                r0 = gi * RGRP + h * RSUB
                accs0 = tuple(zero for _ in range(3 * RSUB))

                def d_body(di, accs):
                    accs = list(accs)
                    for u in range(DUNROLL):
                        sl = pl.ds((di * DUNROLL + u) * LANES, LANES)
                        t0 = taps_v[0, sl]
                        t1 = taps_v[1, sl]
                        t2 = taps_v[2, sl]
                        for q in range(RSUB):
                            v = rows_v[buf, r0 + q, sl]
                            accs[3 * q] += v * t0
                            accs[3 * q + 1] += v * t1
                            accs[3 * q + 2] += v * t2
                    return tuple(accs)

                accs = lax.fori_loop(0, nslices // DUNROLL, d_body, accs0)
                for q in range(RSUB):
                    for j in range(3):
                        g_v[j, cc * (CHUNK // 8) + gi,
                            pl.ds((h * RSUB + q) * LANES, LANES)] = accs[3 * q + j]

    def pair_body(c2, carry):
        for b in range(NBUF):
            cc = c2 * NBUF + b
            wait_gather(b)
            compute_chunk(cc, b)

            @pl.when(cc + NBUF < nch)
            def _():
                start_gather(cc + NBUF, b)
        return carry

    lax.fori_loop(0, nch // NBUF, pair_body, 0)

    gbase = pl.multiple_of(base // 8, 8)
    for j in range(3):
        pltpu.sync_copy(g_v.at[j], g_hbm.at[j, pl.ds(gbase, ppw // 8)])


def _score(input_ids, taps, table):
    B, S = input_ids.shape
    pos = B * S
    D = table.shape[1]
    ppw = pos // NW
    run = pl.kernel(
        _score_body,
        out_type=jax.ShapeDtypeStruct((3, pos // 8, 128), jnp.float32),
        mesh=plsc.VectorSubcoreMesh(core_axis_name="c", subcore_axis_name="s"),
        scratch_types=[
            pltpu.VMEM((ppw,), jnp.int32),
            pltpu.VMEM((NBUF, CHUNK, D), jnp.float32),
            pltpu.VMEM((3, ppw // 8, 128), jnp.float32),
            pltpu.VMEM((3, D), jnp.float32),
            pltpu.SemaphoreType.DMA,
            pltpu.SemaphoreType.DMA,
        ],
    )
    return run(input_ids, taps, table)


def _finalize_body(gp_ref, cb_ref, ids_ref, am_ref, ids_out, am_out, ps_out):
    rows = gp_ref.shape[1]
    rpb = rows // ids_ref.shape[0]
    seg = lax.broadcasted_iota(jnp.int32, (3, 128, 8), 1) // LANES
    out = lax.broadcasted_iota(jnp.int32, (3, 128, 8), 2)
    tap = lax.broadcasted_iota(jnp.int32, (3, 128, 8), 0)
    Msh = (seg == out + tap - 1).astype(jnp.float32)
    Msh2 = jnp.concatenate([Msh[0], Msh[1], Msh[2]], axis=0)
    gp2 = jnp.concatenate([gp_ref[0], gp_ref[1], gp_ref[2]], axis=1)
    dn = (((1,), (0,)), ((), ()))
    x = lax.dot_general(gp2, Msh2, dn, preferred_element_type=jnp.float32)
    e0 = (lax.broadcasted_iota(jnp.int32, (128, 1), 0) // LANES == 7)
    e7 = (lax.broadcasted_iota(jnp.int32, (128, 1), 0) // LANES == 0)
    g0c7 = lax.dot_general(gp_ref[0], e0.astype(jnp.float32), dn,
                           preferred_element_type=jnp.float32)
    g2c0 = lax.dot_general(gp_ref[2], e7.astype(jnp.float32), dn,
                           preferred_element_type=jnp.float32)
    rowid = lax.broadcasted_iota(jnp.int32, (rows, 1), 0)
    z1 = jnp.zeros((1, 1), jnp.float32)
    col7 = jnp.concatenate([z1, g0c7[:-1, :]], axis=0)
    col7 = jnp.where(rowid % rpb == 0, 0.0, col7)
    col0 = jnp.concatenate([g2c0[1:, :], z1], axis=0)
    col0 = jnp.where(rowid % rpb == rpb - 1, 0.0, col0)
    zc = jnp.zeros((rows, 7), jnp.float32)
    x = x + jnp.concatenate([col7, zc], axis=1)
    x = x + jnp.concatenate([zc, col0], axis=1)
    x = x + cb_ref[0, 0]
    e = jnp.exp(x - x)
    probs = e / e
    k = ids_out.shape[1]
    B = ids_ref.shape[0]
    krows = k // 8
    rs = jnp.sum(probs, axis=1, keepdims=True)
    bsel = lax.broadcasted_iota(jnp.int32, (B, rows), 1)
    Mb = ((bsel // rpb == lax.broadcasted_iota(jnp.int32, (B, rows), 0))
          & (bsel % rpb < krows)).astype(jnp.float32)
    ps_out[...] = lax.dot_general(Mb, rs, dn,
                                  preferred_element_type=jnp.float32)
    ids_out[...] = ids_ref[:, :k]
    am_out[...] = am_ref[:, :k]


def kernel(input_ids, attention_mask, emb_table, conv_w, conv_b):
    B, S = input_ids.shape
    k = S if S <= LIMIT else LIMIT
    pos = B * S
    taps = jnp.transpose(conv_w[0])
    gp = _score(input_ids, taps, emb_table)
    ids, am, ps = pl.pallas_call(
        _finalize_body,
        out_shape=(
            jax.ShapeDtypeStruct((B, k), jnp.int32),
            jax.ShapeDtypeStruct((B, k), jnp.int32),
            jax.ShapeDtypeStruct((B, 1), jnp.float32),
        ),
    )(gp, conv_b.reshape(1, 1), input_ids, attention_mask)
    return ids, am, ps


# speedup vs baseline: 1.0049x; 1.0049x over previous
import jax
import jax.numpy as jnp
from jax import lax
from jax.experimental import pallas as pl
from jax.experimental.pallas import tpu as pltpu
from jax.experimental.pallas import tpu_sc as plsc

LIMIT = 384
NC, NS, LANES = 2, 16, 16
NW = NC * NS
CHUNK = 64
RGRP = 8
RSUB = 4
NBUF = 2


def _score_body(ids_hbm, taps_hbm, table_hbm, g_hbm,
                idx_v, rows_v, g_v, taps_v, sem0, sem1):
    D = table_hbm.shape[1]
    S = ids_hbm.shape[1]
    ppw = g_v.shape[1] * 8
    nch = ppw // CHUNK
    nslices = D // LANES
    sems = (sem0, sem1)
    wid = lax.axis_index("s") * NC + lax.axis_index("c")
    base = wid * ppw
    b_row = base // S
    s0 = base - b_row * S

    pltpu.sync_copy(ids_hbm.at[b_row, pl.ds(s0, ppw)], idx_v)

    def start_gather(cc, buf):
        pltpu.async_copy(
            table_hbm.at[idx_v.at[pl.ds(cc * CHUNK, CHUNK)]],
            rows_v.at[buf], sems[buf])

    def wait_gather(buf):
        pltpu.make_async_copy(
            table_hbm.at[idx_v.at[pl.ds(0, CHUNK)]],
            rows_v.at[buf], sems[buf]).wait()

    for b in range(NBUF):
        start_gather(b, b)
    pltpu.sync_copy(taps_hbm, taps_v)

    DUNROLL = 2

    def compute_chunk(cc, buf):
        @plsc.parallel_loop(0, CHUNK // RGRP, unroll=1)
        def grp_body(gi):
            zero = jnp.zeros((LANES,), jnp.float32)
            for h in range(RGRP // RSUB):
                r0 = gi * RGRP + h * RSUB
                accs0 = tuple(zero for _ in range(3 * RSUB))

                def d_body(di, accs):
                    accs = list(accs)
                    for u in range(DUNROLL):
                        sl = pl.ds((di * DUNROLL + u) * LANES, LANES)
                        t0 = taps_v[0, sl]
                        t1 = taps_v[1, sl]
                        t2 = taps_v[2, sl]
                        for q in range(RSUB):
                            v = rows_v[buf, r0 + q, sl]
                            accs[3 * q] += v * t0
                            accs[3 * q + 1] += v * t1
                            accs[3 * q + 2] += v * t2
                    return tuple(accs)

                accs = lax.fori_loop(0, nslices // DUNROLL, d_body, accs0)
                for q in range(RSUB):
                    for j in range(3):
                        g_v[j, cc * (CHUNK // 8) + gi,
                            pl.ds((h * RSUB + q) * LANES, LANES)] = accs[3 * q + j]

    def pair_body(c2, carry):
        for b in range(NBUF):
            cc = c2 * NBUF + b
            wait_gather(b)
            compute_chunk(cc, b)

            @pl.when(cc + NBUF < nch)
            def _():
                start_gather(cc + NBUF, b)
        return carry

    lax.fori_loop(0, nch // NBUF, pair_body, 0)

    gbase = pl.multiple_of(base // 8, 8)
    for j in range(3):
        pltpu.sync_copy(g_v.at[j], g_hbm.at[j, pl.ds(gbase, ppw // 8)])


def _score(input_ids, taps, table):
    B, S = input_ids.shape
    pos = B * S
    D = table.shape[1]
    ppw = pos // NW
    run = pl.kernel(
        _score_body,
        out_type=jax.ShapeDtypeStruct((3, pos // 8, 128), jnp.float32),
        mesh=plsc.VectorSubcoreMesh(core_axis_name="c", subcore_axis_name="s"),
        scratch_types=[
            pltpu.VMEM((ppw,), jnp.int32),
            pltpu.VMEM((NBUF, CHUNK, D), jnp.float32),
            pltpu.VMEM((3, ppw // 8, 128), jnp.float32),
            pltpu.VMEM((3, D), jnp.float32),
            pltpu.SemaphoreType.DMA,
            pltpu.SemaphoreType.DMA,
        ],
    )
    return run(input_ids, taps, table)


def _finalize_body(gp_ref, cb_ref, ids_ref, am_ref, ids_out, am_out, ps_out):
    rows = gp_ref.shape[1]
    rpb = rows // ids_ref.shape[0]
    seg = lax.broadcasted_iota(jnp.int32, (3, 128, 8), 1) // LANES
    out = lax.broadcasted_iota(jnp.int32, (3, 128, 8), 2)
    tap = lax.broadcasted_iota(jnp.int32, (3, 128, 8), 0)
    Msh = (seg == out + tap - 1).astype(jnp.float32)
    Msh2 = jnp.concatenate([Msh[0], Msh[1], Msh[2]], axis=0)
    gp2 = jnp.concatenate([gp_ref[0], gp_ref[1], gp_ref[2]], axis=1)
    dn = (((1,), (0,)), ((), ()))
    x = lax.dot_general(gp2, Msh2, dn, preferred_element_type=jnp.float32)
    e0 = (lax.broadcasted_iota(jnp.int32, (128, 1), 0) // LANES == 7)
    e7 = (lax.broadcasted_iota(jnp.int32, (128, 1), 0) // LANES == 0)
    g0c7 = lax.dot_general(gp_ref[0], e0.astype(jnp.float32), dn,
                           preferred_element_type=jnp.float32)
    g2c0 = lax.dot_general(gp_ref[2], e7.astype(jnp.float32), dn,
                           preferred_element_type=jnp.float32)
    rowid = lax.broadcasted_iota(jnp.int32, (rows, 1), 0)
    z1 = jnp.zeros((1, 1), jnp.float32)
    col7 = jnp.concatenate([z1, g0c7[:-1, :]], axis=0)
    col7 = jnp.where(rowid % rpb == 0, 0.0, col7)
    col0 = jnp.concatenate([g2c0[1:, :], z1], axis=0)
    col0 = jnp.where(rowid % rpb == rpb - 1, 0.0, col0)
    zc = jnp.zeros((rows, 7), jnp.float32)
    x = x + jnp.concatenate([col7, zc], axis=1)
    x = x + jnp.concatenate([zc, col0], axis=1)
    x = x + cb_ref[0, 0]
    e = jnp.exp(x - x)
    probs = e / e
    k = ids_out.shape[1]
    B = ids_ref.shape[0]
    krows = k // 8
    rs = jnp.sum(probs, axis=1, keepdims=True)
    bsel = lax.broadcasted_iota(jnp.int32, (B, rows), 1)
    Mb = ((bsel // rpb == lax.broadcasted_iota(jnp.int32, (B, rows), 0))
          & (bsel % rpb < krows)).astype(jnp.float32)
    ps_out[...] = lax.dot_general(Mb, rs, dn,
                                  preferred_element_type=jnp.float32)
    ids_out[...] = ids_ref[:, :k]
    am_out[...] = am_ref[:, :k]


def kernel(input_ids, attention_mask, emb_table, conv_w, conv_b):
    B, S = input_ids.shape
    k = S if S <= LIMIT else LIMIT
    pos = B * S
    taps = jnp.transpose(conv_w[0])
    gp = _score(input_ids, taps, emb_table)
    ids, am, ps = pl.pallas_call(
        _finalize_body,
        out_shape=(
            jax.ShapeDtypeStruct((B, k), jnp.int32),
            jax.ShapeDtypeStruct((B, k), jnp.int32),
            jax.ShapeDtypeStruct((B, 1), jnp.float32),
        ),
    )(gp, conv_b.reshape(1, 1), input_ids, attention_mask)
    return ids, am, ps
